# Initial kernel scaffold; baseline (speedup 1.0000x reference)
import jax
import jax.numpy as jnp
from jax.experimental import pallas as pl


def kernel(positions, batch):
    raise NotImplementedError("write your pallas kernel here")


# speedup vs baseline: 6.6812x; 6.6812x over previous
import jax
import jax.numpy as jnp
from jax.experimental import pallas as pl

_N = 4096
_K1 = 17
_BR = 256
_GRID = _N // _BR


def _body(p4_ref, post_ref, dist_ref, gath_ref, scat_ref):
    i = pl.program_id(0)
    base = i * _BR

    acc = jnp.zeros((_BR, _N), jnp.float32)
    for d in range(3):
        row = post_ref[d:d + 1, :]
        col = p4_ref[pl.ds(base, _BR), d:d + 1]
        diff = row - col
        acc = acc + diff * diff
    cur = jnp.sqrt(acc + 1e-8)

    iota = jax.lax.broadcasted_iota(jnp.int32, (_BR, _N), 1)
    iota_k = jax.lax.broadcasted_iota(jnp.int32, (_BR, _K1 + 15), 1)

    def step(m, carry):
        cur, w, mins = carry
        mn = jnp.min(cur, axis=1, keepdims=True)
        idx = jnp.min(jnp.where(cur == mn, iota, _N), axis=1, keepdims=True)
        sel = iota == idx
        coef = jnp.where(m > 0, 1.0, 0.0) / mn
        w = w + jnp.where(sel, coef, 0.0)
        mins = mins + jnp.where(iota_k == m, mn, 0.0)
        cur = jnp.where(sel, jnp.inf, cur)
        return cur, w, mins

    carry = (cur,
             jnp.zeros((_BR, _N), jnp.float32),
             jnp.zeros((_BR, _K1 + 15), jnp.float32))
    _, w, mins = jax.lax.fori_loop(0, _K1, step, carry)
    dist_ref[...] = mins[:, 1:_K1]

    p4 = p4_ref[...]
    p4blk = p4_ref[pl.ds(base, _BR), :]
    gath_ref[...] = jnp.dot(w, p4, preferred_element_type=jnp.float32,
                            precision=jax.lax.Precision.HIGHEST)
    contrib = jax.lax.dot_general(
        w, p4blk, (((0,), (0,)), ((), ())), preferred_element_type=jnp.float32,
        precision=jax.lax.Precision.HIGHEST)

    @pl.when(i == 0)
    def _init():
        scat_ref[...] = contrib

    @pl.when(i > 0)
    def _acc():
        scat_ref[...] = scat_ref[...] + contrib


def kernel(positions, batch):
    pos = positions.astype(jnp.float32)
    p4 = jnp.concatenate([pos, jnp.ones((_N, 1), jnp.float32)], axis=1)
    post = pos.T

    dist_o, gath, scat = pl.pallas_call(
        _body,
        grid=(_GRID,),
        in_specs=[
            pl.BlockSpec((_N, 4), lambda i: (0, 0)),
            pl.BlockSpec((3, _N), lambda i: (0, 0)),
        ],
        out_specs=[
            pl.BlockSpec((_BR, _K1 - 1), lambda i: (i, 0)),
            pl.BlockSpec((_BR, 4), lambda i: (i, 0)),
            pl.BlockSpec((_N, 4), lambda i: (0, 0)),
        ],
        out_shape=[
            jax.ShapeDtypeStruct((_N, _K1 - 1), jnp.float32),
            jax.ShapeDtypeStruct((_N, 4), jnp.float32),
            jax.ShapeDtypeStruct((_N, 4), jnp.float32),
        ],
    )(p4, post)

    grad = pos * (gath[:, 3:4] + scat[:, 3:4]) - gath[:, :3] - scat[:, :3]
    return (dist_o.reshape(1, -1), (grad,))
